# baseline (device time: 179543 ns/iter reference)
import jax
import jax.numpy as jnp
from jax import lax
from jax.experimental import pallas as pl
from jax.experimental.pallas import tpu as pltpu

N_DEV = 4
M = 2048
D = 2048
F = 8192
CHUNK = D // N_DEV
FTILE = 1024
NT = F // FTILE
FSUB = 512
NSUB = F // FSUB
SUB_PER_TILE = FTILE // FSUB


def kernel(x, dy):
    x = x.astype(jnp.bfloat16).T

    def body(x_ref, dy_hbm, out_hbm, dybf_hbm, dyf32_buf, stage_buf,
             send_buf, recv_buf, f32_sems, wb_sems, ld_sems, out_sems,
             send_sems, recv_sems):
        my = lax.axis_index("i")
        left = lax.rem(my + N_DEV - 1, N_DEV)
        right = lax.rem(my + 1, N_DEV)

        barrier = pltpu.get_barrier_semaphore()
        for nbr in (left, right):
            pl.semaphore_signal(
                barrier, inc=1,
                device_id=(nbr,), device_id_type=pl.DeviceIdType.MESH,
            )
        pl.semaphore_wait(barrier, 2)

        def f32_copy(u, slot):
            return pltpu.make_async_copy(
                dy_hbm.at[:, pl.ds(u * FSUB, FSUB)],
                dyf32_buf.at[slot],
                f32_sems.at[slot],
            )

        def wb_copy(u, slot):
            return pltpu.make_async_copy(
                stage_buf.at[slot], dybf_hbm.at[u], wb_sems.at[slot],
            )

        def ld_copy(u, slot):
            return pltpu.make_async_copy(
                dybf_hbm.at[u], stage_buf.at[slot], ld_sems.at[slot],
            )

        def out_copy(t):
            return pltpu.make_async_copy(
                send_buf.at[t],
                out_hbm.at[:, pl.ds(t * FTILE, FTILE)],
                out_sems.at[t],
            )

        def tile_rdma(s, t):
            return pltpu.make_async_remote_copy(
                src_ref=send_buf.at[t],
                dst_ref=recv_buf.at[s, t],
                send_sem=send_sems.at[s, t],
                recv_sem=recv_sems.at[s, t],
                device_id=(right if t % 2 == 0 else left,),
                device_id_type=pl.DeviceIdType.MESH,
            )

        def x_slices(c_r, c_l):
            return (x_ref[pl.ds(c_r * CHUNK, CHUNK), :],
                    x_ref[pl.ds(c_l * CHUNK, CHUNK), :])

        xs_r, xs_l = x_slices(lax.rem(my + N_DEV - 1, N_DEV),
                              lax.rem(my + 1, N_DEV))
        f32_copy(0, 0).start()
        for u in range(NSUB):
            t, h = u // SUB_PER_TILE, u % SUB_PER_TILE
            slot = u % 2
            if u + 1 < NSUB:
                f32_copy(u + 1, (u + 1) % 2).start()
            f32_copy(u, slot).wait()
            if u >= 2:
                wb_copy(u - 2, slot).wait()
            bf = dyf32_buf[slot].astype(jnp.bfloat16)
            stage_buf[slot] = bf
            wb_copy(u, slot).start()
            mm = lax.dot_general(
                xs_r if t % 2 == 0 else xs_l, bf,
                dimension_numbers=(((1,), (0,)), ((), ())),
                preferred_element_type=jnp.float32,
            ).astype(jnp.bfloat16)
            send_buf[t, :, pl.ds(h * FSUB, FSUB)] = mm
            if h == SUB_PER_TILE - 1:
                tile_rdma(0, t).start()
        wb_copy(NSUB - 2, 0).wait()
        wb_copy(NSUB - 1, 1).wait()

        ld_copy(0, 0).start()
        for s in range(1, N_DEV):
            if s < N_DEV - 1:
                xs_r, xs_l = x_slices(lax.rem(my + 2 * N_DEV - 1 - s, N_DEV),
                                      lax.rem(my + 1 + s, N_DEV))
            else:
                xs_r, xs_l = x_slices(my, my)
                xs_l = xs_r
            for u in range(NSUB):
                t, h = u // SUB_PER_TILE, u % SUB_PER_TILE
                g = (s - 1) * NSUB + u
                slot = g % 2
                if g + 1 < (N_DEV - 1) * NSUB:
                    ld_copy((u + 1) % NSUB, (g + 1) % 2).start()
                ld_copy(u, slot).wait()
                if h == 0:
                    tile_rdma(s - 1, t).wait_recv()
                    tile_rdma(s - 1, t).wait_send()
                mm = lax.dot_general(
                    xs_r if t % 2 == 0 else xs_l, stage_buf[slot],
                    dimension_numbers=(((1,), (0,)), ((), ())),
                    preferred_element_type=jnp.float32,
                ).astype(jnp.bfloat16)
                hsl = pl.ds(h * FSUB, FSUB)
                send_buf[t, :, hsl] = mm + recv_buf[s - 1, t, :, hsl]
                if h == SUB_PER_TILE - 1:
                    if s < N_DEV - 1:
                        tile_rdma(s, t).start()
                    else:
                        out_copy(t).start()
        for t in range(NT):
            out_copy(t).wait()

    out, _ = pl.pallas_call(
        body,
        out_shape=(
            jax.ShapeDtypeStruct((CHUNK, F), jnp.bfloat16),
            jax.ShapeDtypeStruct((NSUB, M, FSUB), jnp.bfloat16),
        ),
        in_specs=[
            pl.BlockSpec(memory_space=pltpu.VMEM),
            pl.BlockSpec(memory_space=pl.ANY),
        ],
        out_specs=(
            pl.BlockSpec(memory_space=pl.ANY),
            pl.BlockSpec(memory_space=pl.ANY),
        ),
        scratch_shapes=[
            pltpu.VMEM((2, M, FSUB), jnp.float32),
            pltpu.VMEM((2, M, FSUB), jnp.bfloat16),
            pltpu.VMEM((NT, CHUNK, FTILE), jnp.bfloat16),
            pltpu.VMEM((N_DEV - 1, NT, CHUNK, FTILE), jnp.bfloat16),
            pltpu.SemaphoreType.DMA((2,)),
            pltpu.SemaphoreType.DMA((2,)),
            pltpu.SemaphoreType.DMA((2,)),
            pltpu.SemaphoreType.DMA((NT,)),
            pltpu.SemaphoreType.DMA((N_DEV - 1, NT)),
            pltpu.SemaphoreType.DMA((N_DEV - 1, NT)),
        ],
        compiler_params=pltpu.CompilerParams(
            collective_id=0,
            vmem_limit_bytes=63 * 1024 * 1024,
        ),
    )(x, dy)
    return out


# device time: 173631 ns/iter; 1.0340x vs baseline; 1.0340x over previous
import jax
import jax.numpy as jnp
from jax import lax
from jax.experimental import pallas as pl
from jax.experimental.pallas import tpu as pltpu

N_DEV = 4
M = 2048
D = 2048
F = 8192
CHUNK = D // N_DEV
FTILE = 512
NT = F // FTILE
FSUB = 512
NSUB = F // FSUB
SUB_PER_TILE = FTILE // FSUB


def kernel(x, dy):
    x = x.astype(jnp.bfloat16).T

    def body(x_ref, dy_hbm, out_hbm, dybf_hbm, dyf32_buf, stage_buf,
             send_buf, recv_buf, f32_sems, wb_sems, ld_sems, out_sems,
             send_sems, recv_sems):
        my = lax.axis_index("i")
        left = lax.rem(my + N_DEV - 1, N_DEV)
        right = lax.rem(my + 1, N_DEV)

        barrier = pltpu.get_barrier_semaphore()
        for nbr in (left, right):
            pl.semaphore_signal(
                barrier, inc=1,
                device_id=(nbr,), device_id_type=pl.DeviceIdType.MESH,
            )
        pl.semaphore_wait(barrier, 2)

        def f32_copy(u, slot):
            return pltpu.make_async_copy(
                dy_hbm.at[:, pl.ds(u * FSUB, FSUB)],
                dyf32_buf.at[slot],
                f32_sems.at[slot],
            )

        def wb_copy(u, slot):
            return pltpu.make_async_copy(
                stage_buf.at[slot], dybf_hbm.at[u], wb_sems.at[slot],
            )

        def ld_copy(u, slot):
            return pltpu.make_async_copy(
                dybf_hbm.at[u], stage_buf.at[slot], ld_sems.at[slot],
            )

        def out_copy(t):
            return pltpu.make_async_copy(
                send_buf.at[t],
                out_hbm.at[:, pl.ds(t * FTILE, FTILE)],
                out_sems.at[t],
            )

        def tile_rdma(s, t):
            return pltpu.make_async_remote_copy(
                src_ref=send_buf.at[t],
                dst_ref=recv_buf.at[s, t],
                send_sem=send_sems.at[s, t],
                recv_sem=recv_sems.at[s, t],
                device_id=(right if t % 2 == 0 else left,),
                device_id_type=pl.DeviceIdType.MESH,
            )

        def x_slices(c_r, c_l):
            return (x_ref[pl.ds(c_r * CHUNK, CHUNK), :],
                    x_ref[pl.ds(c_l * CHUNK, CHUNK), :])

        xs_r, xs_l = x_slices(lax.rem(my + N_DEV - 1, N_DEV),
                              lax.rem(my + 1, N_DEV))
        f32_copy(0, 0).start()
        for u in range(NSUB):
            t, h = u // SUB_PER_TILE, u % SUB_PER_TILE
            slot = u % 2
            if u + 1 < NSUB:
                f32_copy(u + 1, (u + 1) % 2).start()
            f32_copy(u, slot).wait()
            if u >= 2:
                wb_copy(u - 2, slot).wait()
            bf = dyf32_buf[slot].astype(jnp.bfloat16)
            stage_buf[slot] = bf
            wb_copy(u, slot).start()
            mm = lax.dot_general(
                xs_r if t % 2 == 0 else xs_l, bf,
                dimension_numbers=(((1,), (0,)), ((), ())),
                preferred_element_type=jnp.float32,
            ).astype(jnp.bfloat16)
            send_buf[t, :, pl.ds(h * FSUB, FSUB)] = mm
            if h == SUB_PER_TILE - 1:
                tile_rdma(0, t).start()
        wb_copy(NSUB - 2, 0).wait()
        wb_copy(NSUB - 1, 1).wait()

        ld_copy(0, 0).start()
        for s in range(1, N_DEV):
            if s < N_DEV - 1:
                xs_r, xs_l = x_slices(lax.rem(my + 2 * N_DEV - 1 - s, N_DEV),
                                      lax.rem(my + 1 + s, N_DEV))
            else:
                xs_r, xs_l = x_slices(my, my)
                xs_l = xs_r
            for u in range(NSUB):
                t, h = u // SUB_PER_TILE, u % SUB_PER_TILE
                g = (s - 1) * NSUB + u
                slot = g % 2
                if g + 1 < (N_DEV - 1) * NSUB:
                    ld_copy((u + 1) % NSUB, (g + 1) % 2).start()
                ld_copy(u, slot).wait()
                if h == 0:
                    tile_rdma(s - 1, t).wait_recv()
                    tile_rdma(s - 1, t).wait_send()
                mm = lax.dot_general(
                    xs_r if t % 2 == 0 else xs_l, stage_buf[slot],
                    dimension_numbers=(((1,), (0,)), ((), ())),
                    preferred_element_type=jnp.float32,
                ).astype(jnp.bfloat16)
                hsl = pl.ds(h * FSUB, FSUB)
                send_buf[t, :, hsl] = mm + recv_buf[s - 1, t, :, hsl]
                if h == SUB_PER_TILE - 1:
                    if s < N_DEV - 1:
                        tile_rdma(s, t).start()
                    else:
                        out_copy(t).start()
        for t in range(NT):
            out_copy(t).wait()

    out, _ = pl.pallas_call(
        body,
        out_shape=(
            jax.ShapeDtypeStruct((CHUNK, F), jnp.bfloat16),
            jax.ShapeDtypeStruct((NSUB, M, FSUB), jnp.bfloat16),
        ),
        in_specs=[
            pl.BlockSpec(memory_space=pltpu.VMEM),
            pl.BlockSpec(memory_space=pl.ANY),
        ],
        out_specs=(
            pl.BlockSpec(memory_space=pl.ANY),
            pl.BlockSpec(memory_space=pl.ANY),
        ),
        scratch_shapes=[
            pltpu.VMEM((2, M, FSUB), jnp.float32),
            pltpu.VMEM((2, M, FSUB), jnp.bfloat16),
            pltpu.VMEM((NT, CHUNK, FTILE), jnp.bfloat16),
            pltpu.VMEM((N_DEV - 1, NT, CHUNK, FTILE), jnp.bfloat16),
            pltpu.SemaphoreType.DMA((2,)),
            pltpu.SemaphoreType.DMA((2,)),
            pltpu.SemaphoreType.DMA((2,)),
            pltpu.SemaphoreType.DMA((NT,)),
            pltpu.SemaphoreType.DMA((N_DEV - 1, NT)),
            pltpu.SemaphoreType.DMA((N_DEV - 1, NT)),
        ],
        compiler_params=pltpu.CompilerParams(
            collective_id=0,
            vmem_limit_bytes=63 * 1024 * 1024,
        ),
    )(x, dy)
    return out
